# Initial kernel scaffold; baseline (speedup 1.0000x reference)
import jax
import jax.numpy as jnp
from jax.experimental import pallas as pl


def kernel(x, boxes, scores, W1, b1, W2, b2, Wb, bb, Wc, bc):
    raise NotImplementedError("write your pallas kernel here")


# speedup vs baseline: 1.3260x; 1.3260x over previous
import jax
import jax.numpy as jnp
from jax import lax
from jax.experimental import pallas as pl
from jax.experimental.pallas import tpu as pltpu
from jax.experimental.pallas import tpu_sc as plsc

L = 16
NCHUNK = 7
NPAD = NCHUNK * L
NBOX = 100
NIMG = 8
IOU_THR = 0.5


def _nms_body(sc_hbm, x1_hbm, y1_hbm, x2_hbm, y2_hbm,
              os_hbm, ox1_hbm, oy1_hbm, ox2_hbm, oy2_hbm, cnt_hbm,
              s_v, x1_v, y1_v, x2_v, y2_v,
              ss_v, sx1_v, sy1_v, sx2_v, sy2_v, area_v, keep_v, cnt_v):
    wid = lax.axis_index("s") * 2 + lax.axis_index("c")

    @pl.when(wid < NIMG)
    def _():
        pltpu.sync_copy(sc_hbm.at[wid], s_v)
        pltpu.sync_copy(x1_hbm.at[wid], x1_v)
        pltpu.sync_copy(y1_hbm.at[wid], y1_v)
        pltpu.sync_copy(x2_hbm.at[wid], x2_v)
        pltpu.sync_copy(y2_hbm.at[wid], y2_v)

        iota = lax.iota(jnp.int32, L)
        gidx = [iota + iv * L for iv in range(NCHUNK)]
        svecs = [s_v[pl.ds(iv * L, L)] for iv in range(NCHUNK)]

        zeros = jnp.zeros((L,), jnp.int32)

        def rank_body(j, ranks):
            sj = plsc.load_gather(s_v, [zeros + j])
            out = []
            for iv in range(NCHUNK):
                beats = (sj > svecs[iv]) | ((sj == svecs[iv]) & (j < gidx[iv]))
                out.append(ranks[iv] + beats.astype(jnp.int32))
            return tuple(out)

        ranks = lax.fori_loop(0, NPAD, rank_body,
                              tuple(zeros for _ in range(NCHUNK)))

        for iv in range(NCHUNK):
            sl = pl.ds(iv * L, L)
            r = ranks[iv]
            plsc.store_scatter(ss_v, [r], svecs[iv])
            plsc.store_scatter(sx1_v, [r], x1_v[sl])
            plsc.store_scatter(sy1_v, [r], y1_v[sl])
            plsc.store_scatter(sx2_v, [r], x2_v[sl])
            plsc.store_scatter(sy2_v, [r], y2_v[sl])

        ones = jnp.ones((L,), jnp.int32)
        for iv in range(NCHUNK):
            sl = pl.ds(iv * L, L)
            w = jnp.maximum(sx2_v[sl] - sx1_v[sl], 0.0)
            h = jnp.maximum(sy2_v[sl] - sy1_v[sl], 0.0)
            area_v[sl] = w * h
            keep_v[sl] = ones

        def nms_body(i, carry):
            isplat = zeros + i
            alive = plsc.load_gather(keep_v, [isplat]) != 0
            xi1 = plsc.load_gather(sx1_v, [isplat])
            yi1 = plsc.load_gather(sy1_v, [isplat])
            xi2 = plsc.load_gather(sx2_v, [isplat])
            yi2 = plsc.load_gather(sy2_v, [isplat])
            ai = plsc.load_gather(area_v, [isplat])
            for jv in range(NCHUNK):
                @pl.when(jv * L + (L - 1) > i)
                def _(jv=jv):
                    sl = pl.ds(jv * L, L)
                    xx1 = jnp.maximum(sx1_v[sl], xi1)
                    yy1 = jnp.maximum(sy1_v[sl], yi1)
                    xx2 = jnp.minimum(sx2_v[sl], xi2)
                    yy2 = jnp.minimum(sy2_v[sl], yi2)
                    inter = (jnp.maximum(xx2 - xx1, 0.0) *
                             jnp.maximum(yy2 - yy1, 0.0))
                    union = ai + area_v[sl] - inter
                    iou = inter / jnp.maximum(union, 1e-9)
                    supp = (iou > IOU_THR) & (gidx[jv] > i) & alive
                    keep_v[sl] = jnp.where(supp, 0, keep_v[sl])
            return carry

        lax.fori_loop(0, NBOX, nms_body, 0)

        total = jnp.int32(0)
        for iv in range(NCHUNK):
            sl = pl.ds(iv * L, L)
            kv = keep_v[sl]
            total = total + jnp.sum(kv * (gidx[iv] < NBOX).astype(jnp.int32))
            kf = kv.astype(jnp.float32)
            ss_v[sl] = ss_v[sl] * kf
            sx1_v[sl] = sx1_v[sl] * kf
            sy1_v[sl] = sy1_v[sl] * kf
            sx2_v[sl] = sx2_v[sl] * kf
            sy2_v[sl] = sy2_v[sl] * kf
        cnt_v[...] = zeros + total

        pltpu.sync_copy(ss_v, os_hbm.at[wid])
        pltpu.sync_copy(sx1_v, ox1_hbm.at[wid])
        pltpu.sync_copy(sy1_v, oy1_hbm.at[wid])
        pltpu.sync_copy(sx2_v, ox2_hbm.at[wid])
        pltpu.sync_copy(sy2_v, oy2_hbm.at[wid])
        pltpu.sync_copy(cnt_v, cnt_hbm.at[wid])


def kernel(x, boxes, scores, W1, b1, W2, b2, Wb, bb, Wc, bc):
    del x, W1, b1, W2, b2, Wb, bb, Wc, bc
    nb, nn = scores.shape
    pad = NPAD - nn
    sc_p = jnp.pad(scores, ((0, 0), (0, pad)), constant_values=-1.0)
    bx_p = jnp.pad(boxes, ((0, 0), (0, pad), (0, 0)))
    x1, y1, x2, y2 = (bx_p[:, :, k] for k in range(4))

    mesh = plsc.VectorSubcoreMesh(core_axis_name="c", subcore_axis_name="s",
                                  num_cores=2, num_subcores=16)
    f32 = jnp.float32
    vec_f32 = pltpu.VMEM((NPAD,), f32)
    outs = pl.kernel(
        _nms_body,
        out_type=(
            jax.ShapeDtypeStruct((nb, NPAD), f32),
            jax.ShapeDtypeStruct((nb, NPAD), f32),
            jax.ShapeDtypeStruct((nb, NPAD), f32),
            jax.ShapeDtypeStruct((nb, NPAD), f32),
            jax.ShapeDtypeStruct((nb, NPAD), f32),
            jax.ShapeDtypeStruct((nb, L), jnp.int32),
        ),
        mesh=mesh,
        compiler_params=pltpu.CompilerParams(needs_layout_passes=False),
        scratch_types=[vec_f32] * 11 + [pltpu.VMEM((NPAD,), jnp.int32),
                                        pltpu.VMEM((L,), jnp.int32)],
    )(sc_p, x1, y1, x2, y2)

    os_, ox1, oy1, ox2, oy2, cnt = outs
    final_scores = os_[:, :nn]
    final_boxes = jnp.stack([ox1[:, :nn], oy1[:, :nn],
                             ox2[:, :nn], oy2[:, :nn]], axis=-1)
    num_detections = cnt[:, 0]
    return final_boxes, final_scores, num_detections
